# Initial kernel scaffold; baseline (speedup 1.0000x reference)
import jax
import jax.numpy as jnp
from jax.experimental import pallas as pl


def kernel(indices, object_table):
    raise NotImplementedError("write your pallas kernel here")


# speedup vs baseline: 2.2621x; 2.2621x over previous
import jax
import jax.numpy as jnp
from jax import lax
from jax.experimental import pallas as pl
from jax.experimental.pallas import tpu as pltpu
from jax.experimental.pallas import tpu_sc as plsc

_NUM_CORES = 2
_NUM_SUBCORES = 16
_NUM_WORKERS = _NUM_CORES * _NUM_SUBCORES

_ROWS_PER_BLK = 8000


def _score_body(char_ref, tbl_ref, out_ref):
    c = char_ref[0, :]
    x = tbl_ref[...]
    out_ref[0, 0, :] = jnp.sum(x * c[None, :], axis=1)


def _compute_scores(object_table):
    v, d = object_table.shape
    rows = _ROWS_PER_BLK
    nblk = v // rows
    char = lax.slice(object_table, (0, 0), (1, d))
    out = pl.pallas_call(
        _score_body,
        grid=(nblk,),
        in_specs=[
            pl.BlockSpec((1, d), lambda i: (0, 0)),
            pl.BlockSpec((rows, d), lambda i: (i, 0)),
        ],
        out_specs=pl.BlockSpec((1, 1, rows), lambda i: (i, 0, 0)),
        out_shape=jax.ShapeDtypeStruct((nblk, 1, rows), jnp.float32),
    )(char, object_table)
    return out.reshape(v)


def _gather_body(per_w, scores_hbm, idx_hbm, out_hbm, idx_v, out_v, sem):
    wid = lax.axis_index("s") * _NUM_CORES + lax.axis_index("c")
    base = wid * per_w
    pltpu.sync_copy(idx_hbm.at[pl.ds(base, per_w)], idx_v)
    pltpu.async_copy(scores_hbm.at[idx_v], out_v, sem).wait()
    pltpu.sync_copy(out_v, out_hbm.at[pl.ds(base, per_w)])


def _gather_scores(scores, idx_flat):
    n = idx_flat.shape[0]
    per_w = n // _NUM_WORKERS
    mesh = plsc.VectorSubcoreMesh(
        core_axis_name="c", subcore_axis_name="s",
        num_cores=_NUM_CORES, num_subcores=_NUM_SUBCORES)

    def body(scores_hbm, idx_hbm, out_hbm, idx_v, out_v, sem):
        _gather_body(per_w, scores_hbm, idx_hbm, out_hbm, idx_v, out_v, sem)

    f = pl.kernel(
        body,
        mesh=mesh,
        out_type=jax.ShapeDtypeStruct((n,), jnp.float32),
        scratch_types=[
            pltpu.VMEM((per_w,), jnp.int32),
            pltpu.VMEM((per_w,), jnp.float32),
            pltpu.SemaphoreType.DMA,
        ],
    )
    return f(scores, idx_flat)


def kernel(indices, object_table):
    b, l = indices.shape
    scores = _compute_scores(object_table)
    out = _gather_scores(scores, indices.reshape(-1))
    return out.reshape(b, l)
